# Initial kernel scaffold; baseline (speedup 1.0000x reference)
import jax
import jax.numpy as jnp
from jax.experimental import pallas as pl


def kernel(hidden_states, W_down, b_down, rms_w, W_r1, b_r1, W_r2, b_r2, W_r3, w_gate, w_up, w_out):
    raise NotImplementedError("write your pallas kernel here")


# speedup vs baseline: 1.5804x; 1.5804x over previous
import functools
import jax
import jax.numpy as jnp
from jax import lax
from jax.experimental import pallas as pl
from jax.experimental.pallas import tpu as pltpu

T = 2048
H = 2048
D = 256
E = 8
I = 2048

IBLK = 256


def _gelu_exact(x):
    return x * 0.5 * (1.0 + lax.erf(x * (2.0 ** -0.5)))


def _router_body(x_ref, wd_ref, bd_ref, rmsw_ref, w1_ref, b1_ref, w2_ref,
                 b2_ref, w3_ref, hs_ref, comb_ref):
    x = x_ref[...]
    hs = jnp.dot(x, wd_ref[...], preferred_element_type=jnp.float32)
    hs = hs + bd_ref[...][None, :]
    hs_ref[...] = hs
    ms = jnp.mean(hs * hs, axis=-1, keepdims=True)
    hsn = hs * lax.rsqrt(ms + 1e-5) * rmsw_ref[...][None, :]
    z = _gelu_exact(jnp.dot(hsn, w1_ref[...], preferred_element_type=jnp.float32)
                    + b1_ref[...][None, :])
    z = _gelu_exact(jnp.dot(z, w2_ref[...], preferred_element_type=jnp.float32)
                    + b2_ref[...][None, :])
    logits = jnp.dot(z, w3_ref[...], preferred_element_type=jnp.float32)
    m = jnp.max(logits, axis=-1, keepdims=True)
    ex = jnp.exp(logits - m)
    probs = ex / jnp.sum(ex, axis=-1, keepdims=True)
    eidx = lax.broadcasted_iota(jnp.int32, probs.shape, 1)
    m1 = jnp.max(probs, axis=-1, keepdims=True)
    i1 = jnp.min(jnp.where(probs == m1, eidx, E), axis=-1, keepdims=True)
    sel1 = eidx == i1
    masked = jnp.where(sel1, -jnp.inf, probs)
    m2 = jnp.max(masked, axis=-1, keepdims=True)
    i2 = jnp.min(jnp.where(masked == m2, eidx, E), axis=-1, keepdims=True)
    sel2 = eidx == i2
    comb_ref[...] = jnp.where(sel1 | sel2, probs, 0.0)


def _expert_body(x_ref, comb_ref, wg_ref, wu_ref, wo_ref, out_ref):
    e = pl.program_id(0)
    c = pl.program_id(1)

    @pl.when((e == 0) & (c == 0))
    def _():
        out_ref[...] = jnp.zeros_like(out_ref)

    x = x_ref[...]
    g = jnp.dot(x, wg_ref[0], preferred_element_type=jnp.float32)
    u = jnp.dot(x, wu_ref[0], preferred_element_type=jnp.float32)
    h = (g * jax.nn.sigmoid(g)) * u
    comb = comb_ref[...]
    ecol = lax.broadcasted_iota(jnp.int32, comb.shape, 1)
    h = h * jnp.sum(jnp.where(ecol == e, comb, 0.0), axis=1, keepdims=True)
    out_ref[...] += jnp.dot(h, wo_ref[0], preferred_element_type=jnp.float32)


@jax.jit
def kernel(hidden_states, W_down, b_down, rms_w, W_r1, b_r1, W_r2, b_r2,
           W_r3, w_gate, w_up, w_out):
    hs, comb = pl.pallas_call(
        _router_body,
        out_shape=(
            jax.ShapeDtypeStruct((T, D), jnp.float32),
            jax.ShapeDtypeStruct((T, E), jnp.float32),
        ),
    )(hidden_states, W_down, b_down, rms_w, W_r1, b_r1, W_r2, b_r2, W_r3)

    nI = I // IBLK
    out = pl.pallas_call(
        _expert_body,
        grid=(E, nI),
        in_specs=[
            pl.BlockSpec((T, H), lambda e, c: (0, 0)),
            pl.BlockSpec((T, E), lambda e, c: (0, 0)),
            pl.BlockSpec((1, H, IBLK), lambda e, c: (e, 0, c)),
            pl.BlockSpec((1, H, IBLK), lambda e, c: (e, 0, c)),
            pl.BlockSpec((1, IBLK, H), lambda e, c: (e, c, 0)),
        ],
        out_specs=pl.BlockSpec((T, H), lambda e, c: (0, 0)),
        out_shape=jax.ShapeDtypeStruct((T, H), jnp.float32),
    )(hidden_states, comb, w_gate, w_up, w_out)
    return (out, hs)
